# baseline (speedup 1.0000x reference)
import functools

import jax
import jax.numpy as jnp
from jax import lax
from jax.experimental import pallas as pl
from jax.experimental.pallas import tpu as pltpu
from jax.experimental.pallas import tpu_sc as plsc

N = 10000
D = 128
H = 16
C = 40
E = 320000

NC = 2
NS = 16
NW = NC * NS
NPAD = 10240
SLICE = NPAD // NS
EPT = E // NW
WIN = 2000

_mesh = plsc.VectorSubcoreMesh(core_axis_name="c", subcore_axis_name="s")


@functools.partial(
    pl.kernel,
    out_type=jax.ShapeDtypeStruct((NC, NPAD), jnp.float32),
    mesh=_mesh,
    scratch_types=[
        pltpu.VMEM((EPT,), jnp.int32),
        pltpu.VMEM((EPT,), jnp.float32),
        pltpu.VMEM((SLICE,), jnp.float32),
        pltpu.VMEM_SHARED((NPAD,), jnp.float32),
    ],
)
def _sc_deg(col_hbm, ew_hbm, out_hbm, idx_v, val_v, zero_v, sdeg):
    c = lax.axis_index("c")
    s = lax.axis_index("s")
    wid = s * NC + c
    base = wid * EPT

    def zset(i, _):
        zero_v[pl.ds(i * 16, 16)] = jnp.zeros((16,), jnp.float32)
        return 0

    lax.fori_loop(0, SLICE // 16, zset, 0, unroll=4)
    pltpu.sync_copy(zero_v, sdeg.at[pl.ds(s * SLICE, SLICE)])
    plsc.subcore_barrier()

    pltpu.sync_copy(col_hbm.at[pl.ds(base, EPT)], idx_v)
    pltpu.sync_copy(ew_hbm.at[pl.ds(base, EPT)], val_v)
    pltpu.sync_copy(val_v, sdeg.at[idx_v], add=True)
    plsc.subcore_barrier()
    pltpu.sync_copy(sdeg.at[pl.ds(s * SLICE, SLICE)],
                    out_hbm.at[c, pl.ds(s * SLICE, SLICE)])


@functools.partial(
    pl.kernel,
    out_type=jax.ShapeDtypeStruct((NC, NPAD, H), jnp.float32),
    mesh=_mesh,
    scratch_types=[
        pltpu.VMEM((2, WIN), jnp.int32),
        pltpu.VMEM((2, WIN), jnp.int32),
        pltpu.VMEM((2, WIN), jnp.float32),
        pltpu.VMEM((2, WIN, H), jnp.float32),
        pltpu.VMEM((NPAD,), jnp.float32),
        pltpu.VMEM((NPAD,), jnp.float32),
        pltpu.VMEM((SLICE, H), jnp.float32),
        pltpu.VMEM_SHARED((NPAD, H), jnp.float32),
        pltpu.VMEM_SHARED((NPAD, H), jnp.float32),
        pltpu.SemaphoreType.DMA,
        pltpu.SemaphoreType.DMA,
        pltpu.SemaphoreType.DMA,
        pltpu.SemaphoreType.DMA,
    ],
    compiler_params=pltpu.CompilerParams(needs_layout_passes=False,
                                         use_tc_tiling_on_sc=False),
)
def _sc_msg(row_hbm, col_hbm, ew_hbm, h_hbm, degp_hbm, out_hbm,
            rbuf, cbuf, wbuf, msg, dis_v, deg2_v, zero_v, sacc, sh,
            gsem0, gsem1, ssem0, ssem1):
    c = lax.axis_index("c")
    s = lax.axis_index("s")
    wid = s * NC + c
    ebase = wid * EPT
    NWIN = EPT // WIN
    gsem = (gsem0, gsem1)
    ssem = (ssem0, ssem1)

    def zset(i, _):
        zero_v[i, :] = jnp.zeros((16,), jnp.float32)
        return 0

    lax.fori_loop(0, SLICE, zset, 0, unroll=4)
    pltpu.sync_copy(zero_v, sacc.at[pl.ds(s * SLICE, SLICE)])
    pltpu.sync_copy(h_hbm.at[pl.ds(s * SLICE, SLICE)], zero_v)
    pltpu.sync_copy(zero_v, sh.at[pl.ds(s * SLICE, SLICE)])
    pltpu.sync_copy(degp_hbm.at[0], dis_v)
    pltpu.sync_copy(degp_hbm.at[1], deg2_v)
    plsc.subcore_barrier()

    def load_and_gather(w):
        b = w % 2
        base = ebase + w * WIN
        pltpu.sync_copy(row_hbm.at[pl.ds(base, WIN)], rbuf.at[b])
        pltpu.sync_copy(col_hbm.at[pl.ds(base, WIN)], cbuf.at[b])
        pltpu.sync_copy(ew_hbm.at[pl.ds(base, WIN)], wbuf.at[b])
        return pltpu.async_copy(sh.at[rbuf.at[b]], msg.at[b], gsem[b])

    gd = [None, None]
    sd = [None, None]
    gd[0] = load_and_gather(0)

    def newt(i, _):
        dv = dis_v[pl.ds(i * 16, 16)] + deg2_v[pl.ds(i * 16, 16)] + 1.0
        iv = plsc.bitcast(dv, jnp.uint32)
        iv = jnp.uint32(0x5F3759DF) - (iv >> jnp.uint32(1))
        y = plsc.bitcast(iv, jnp.float32)
        x2 = dv * 0.5
        y = y * (1.5 - x2 * y * y)
        y = y * (1.5 - x2 * y * y)
        y = y * (1.5 - x2 * y * y)
        dis_v[pl.ds(i * 16, 16)] = y
        return 0

    lax.fori_loop(0, NPAD // 16, newt, 0, unroll=4)

    for w in range(NWIN):
        b = w % 2
        nb = (w + 1) % 2
        if w + 1 < NWIN:
            if sd[nb] is not None:
                sd[nb].wait()
                sd[nb] = None
            gd[nb] = load_and_gather(w + 1)
        gd[b].wait()

        def scl(k, _):
            rv = rbuf[b, pl.ds(k * 16, 16)]
            dv = plsc.load_gather(dis_v, [rv])
            wbuf[b, pl.ds(k * 16, 16)] = wbuf[b, pl.ds(k * 16, 16)] * dv
            return 0

        lax.fori_loop(0, WIN // 16, scl, 0, unroll=4)

        def row_scale(k, _):
            wv = wbuf[b, pl.ds(k * 16, 16)]
            base16 = k * 16
            for j in range(16):
                msg[b, base16 + j, :] = msg[b, base16 + j, :] * wv[j]
            return 0

        lax.fori_loop(0, WIN // 16, row_scale, 0, unroll=2)

        sd[b] = pltpu.async_copy(msg.at[b], sacc.at[cbuf.at[b]], ssem[b],
                                 add=True)

    for d in sd:
        if d is not None:
            d.wait()
    plsc.subcore_barrier()
    pltpu.sync_copy(sacc.at[pl.ds(s * SLICE, SLICE)],
                    out_hbm.at[c, pl.ds(s * SLICE, SLICE)])


def _tc_h_body(x_ref, w_ref, h_ref):
    h_ref[:N, :] = jnp.dot(x_ref[...], w_ref[...],
                           preferred_element_type=jnp.float32)


_tc_h = pl.pallas_call(
    _tc_h_body,
    out_shape=jax.ShapeDtypeStruct((NPAD, H), jnp.float32),
)


def _tc2_body(acc_ref, degp_ref, h_ref, bg_ref, wo_ref, bo_ref, out_ref):
    deg = degp_ref[0, :N, :] + degp_ref[1, :N, :] + 1.0
    dis = jnp.where(deg > 0.0, lax.rsqrt(jnp.maximum(deg, 1e-30)), 0.0)
    agg = (acc_ref[0, :N, :] + acc_ref[1, :N, :]) * dis
    self_term = h_ref[:N, :] * (dis * dis)
    emb = jax.nn.relu(agg + self_term + bg_ref[...])
    out_ref[...] = jnp.dot(emb, wo_ref[...],
                           preferred_element_type=jnp.float32) + bo_ref[...]


_tc2 = pl.pallas_call(
    _tc2_body,
    out_shape=jax.ShapeDtypeStruct((N, C), jnp.float32),
)


def kernel(x, edge_index, edge_weight, W_gcn, b_gcn, W_out, b_out):
    row = edge_index[0].astype(jnp.int32)
    col = edge_index[1].astype(jnp.int32)
    ew = edge_weight.astype(jnp.float32)

    degp = _sc_deg(col, ew)
    h = _tc_h(x, W_gcn)
    accp = _sc_msg(row, col, ew, h, degp)
    out = _tc2(accp, degp.reshape(NC, NPAD, 1), h,
               b_gcn.reshape(1, H), W_out, b_out.reshape(1, C))
    return out


# speedup vs baseline: 1.1892x; 1.1892x over previous
import functools

import jax
import jax.numpy as jnp
from jax import lax
from jax.experimental import pallas as pl
from jax.experimental.pallas import tpu as pltpu
from jax.experimental.pallas import tpu_sc as plsc

N = 10000
D = 128
H = 16
C = 40
E = 320000

NC = 2
NS = 16
NW = NC * NS
NPAD = 10240
SLICE = NPAD // NS
EPT = E // NW
WIN = 2000

_mesh = plsc.VectorSubcoreMesh(core_axis_name="c", subcore_axis_name="s")


@functools.partial(
    pl.kernel,
    out_type=jax.ShapeDtypeStruct((NC, NPAD), jnp.float32),
    mesh=_mesh,
    scratch_types=[
        pltpu.VMEM((EPT,), jnp.int32),
        pltpu.VMEM((EPT,), jnp.float32),
        pltpu.VMEM((SLICE,), jnp.float32),
        pltpu.VMEM_SHARED((NPAD,), jnp.float32),
    ],
)
def _sc_deg(col_hbm, ew_hbm, out_hbm, idx_v, val_v, zero_v, sdeg):
    c = lax.axis_index("c")
    s = lax.axis_index("s")
    wid = s * NC + c
    base = wid * EPT

    def zset(i, _):
        zero_v[pl.ds(i * 16, 16)] = jnp.zeros((16,), jnp.float32)
        return 0

    lax.fori_loop(0, SLICE // 16, zset, 0, unroll=4)
    pltpu.sync_copy(zero_v, sdeg.at[pl.ds(s * SLICE, SLICE)])
    plsc.subcore_barrier()

    pltpu.sync_copy(col_hbm.at[pl.ds(base, EPT)], idx_v)
    pltpu.sync_copy(ew_hbm.at[pl.ds(base, EPT)], val_v)
    pltpu.sync_copy(val_v, sdeg.at[idx_v], add=True)
    plsc.subcore_barrier()
    pltpu.sync_copy(sdeg.at[pl.ds(s * SLICE, SLICE)],
                    out_hbm.at[c, pl.ds(s * SLICE, SLICE)])


@functools.partial(
    pl.kernel,
    out_type=jax.ShapeDtypeStruct((NC, NPAD, H), jnp.float32),
    mesh=_mesh,
    scratch_types=[
        pltpu.VMEM((2, WIN), jnp.int32),
        pltpu.VMEM((2, WIN), jnp.int32),
        pltpu.VMEM((2, WIN), jnp.float32),
        pltpu.VMEM((2, WIN, H), jnp.float32),
        pltpu.VMEM((NPAD,), jnp.float32),
        pltpu.VMEM((SLICE, H), jnp.float32),
        pltpu.VMEM_SHARED((NPAD, H), jnp.float32),
        pltpu.VMEM_SHARED((NPAD, H), jnp.float32),
        pltpu.SemaphoreType.DMA,
        pltpu.SemaphoreType.DMA,
        pltpu.SemaphoreType.DMA,
        pltpu.SemaphoreType.DMA,
    ],
    compiler_params=pltpu.CompilerParams(needs_layout_passes=False,
                                         use_tc_tiling_on_sc=False),
)
def _sc_msg(row_hbm, col_hbm, ew_hbm, h_hbm, dis_hbm, out_hbm,
            rbuf, cbuf, wbuf, msg, dis_v, zero_v, sacc, sh,
            gsem0, gsem1, ssem0, ssem1):
    c = lax.axis_index("c")
    s = lax.axis_index("s")
    wid = s * NC + c
    ebase = wid * EPT
    NWIN = EPT // WIN
    gsem = (gsem0, gsem1)
    ssem = (ssem0, ssem1)

    def zset(i, _):
        zero_v[i, :] = jnp.zeros((16,), jnp.float32)
        return 0

    lax.fori_loop(0, SLICE, zset, 0, unroll=4)
    pltpu.sync_copy(zero_v, sacc.at[pl.ds(s * SLICE, SLICE)])
    pltpu.sync_copy(h_hbm.at[pl.ds(s * SLICE, SLICE)], zero_v)
    pltpu.sync_copy(zero_v, sh.at[pl.ds(s * SLICE, SLICE)])
    pltpu.sync_copy(dis_hbm, dis_v)
    plsc.subcore_barrier()

    def load_and_gather(w):
        b = w % 2
        base = ebase + w * WIN
        pltpu.sync_copy(row_hbm.at[pl.ds(base, WIN)], rbuf.at[b])
        pltpu.sync_copy(col_hbm.at[pl.ds(base, WIN)], cbuf.at[b])
        pltpu.sync_copy(ew_hbm.at[pl.ds(base, WIN)], wbuf.at[b])
        return pltpu.async_copy(sh.at[rbuf.at[b]], msg.at[b], gsem[b])

    gd = [None, None]
    sd = [None, None]
    gd[0] = load_and_gather(0)

    for w in range(NWIN):
        b = w % 2
        nb = (w + 1) % 2
        if w + 1 < NWIN:
            if sd[nb] is not None:
                sd[nb].wait()
                sd[nb] = None
            gd[nb] = load_and_gather(w + 1)
        gd[b].wait()

        @plsc.parallel_loop(0, WIN // 16, unroll=4)
        def scl(k):
            rv = rbuf[b, pl.ds(k * 16, 16)]
            dv = plsc.load_gather(dis_v, [rv])
            wbuf[b, pl.ds(k * 16, 16)] = wbuf[b, pl.ds(k * 16, 16)] * dv

        @plsc.parallel_loop(0, WIN // 16, unroll=2)
        def row_scale(k):
            wv = wbuf[b, pl.ds(k * 16, 16)]
            base16 = k * 16
            for j in range(16):
                msg[b, base16 + j, :] = msg[b, base16 + j, :] * wv[j]

        sd[b] = pltpu.async_copy(msg.at[b], sacc.at[cbuf.at[b]], ssem[b],
                                 add=True)

    for d in sd:
        if d is not None:
            d.wait()
    plsc.subcore_barrier()
    pltpu.sync_copy(sacc.at[pl.ds(s * SLICE, SLICE)],
                    out_hbm.at[c, pl.ds(s * SLICE, SLICE)])


def _tc1_body(x_ref, w_ref, degp_ref, h_ref, dis_ref):
    h_ref[:N, :] = jnp.dot(x_ref[...], w_ref[...],
                           preferred_element_type=jnp.float32)
    deg = degp_ref[0, :] + degp_ref[1, :] + 1.0
    dis_ref[...] = jnp.where(deg > 0.0,
                             lax.rsqrt(jnp.maximum(deg, 1e-30)),
                             0.0)


_tc1 = pl.pallas_call(
    _tc1_body,
    out_shape=(
        jax.ShapeDtypeStruct((NPAD, H), jnp.float32),
        jax.ShapeDtypeStruct((NPAD,), jnp.float32),
    ),
)


def _tc2_body(acc_ref, dis_ref, h_ref, bg_ref, wo_ref, bo_ref, out_ref):
    dis = dis_ref[:N, :]
    agg = (acc_ref[0, :N, :] + acc_ref[1, :N, :]) * dis
    self_term = h_ref[:N, :] * (dis * dis)
    emb = jax.nn.relu(agg + self_term + bg_ref[...])
    out_ref[...] = jnp.dot(emb, wo_ref[...],
                           preferred_element_type=jnp.float32) + bo_ref[...]


_tc2 = pl.pallas_call(
    _tc2_body,
    out_shape=jax.ShapeDtypeStruct((N, C), jnp.float32),
)


def kernel(x, edge_index, edge_weight, W_gcn, b_gcn, W_out, b_out):
    row = edge_index[0].astype(jnp.int32)
    col = edge_index[1].astype(jnp.int32)
    ew = edge_weight.astype(jnp.float32)

    degp = _sc_deg(col, ew)
    h, dis = _tc1(x, W_gcn, degp)
    accp = _sc_msg(row, col, ew, h, dis)
    out = _tc2(accp, dis.reshape(NPAD, 1), h,
               b_gcn.reshape(1, H), W_out, b_out.reshape(1, C))
    return out
